# baseline (speedup 1.0000x reference)
import functools

import jax
import jax.numpy as jnp
from jax.experimental import pallas as pl
from jax.experimental.pallas import tpu as pltpu

_TRIM = 0.2


_ROWS = 4


def _tree_sum(x):
    ones8 = jnp.full((8, x.shape[0]), 0.125, jnp.float32)
    partial = jax.lax.dot_general(
        ones8, x, (((1,), (0,)), ((), ())),
        preferred_element_type=jnp.float32,
    )
    return jnp.sum(partial)


def _row_kernel(pred_ref, tgt_ref, out_ref, acc_ref):
    b = pl.program_id(0)
    nb = pl.num_programs(0)
    h = pred_ref.shape[1]
    w = pred_ref.shape[2]
    n = h * w

    @pl.when(b == 0)
    def _init():
        acc_ref[0] = 0.0

    loss_sum = acc_ref[0]
    cnts = []
    for r in range(_ROWS):
        cnts.append(
            _tree_sum((tgt_ref[r] <= 0).astype(jnp.float32)).astype(jnp.int32)
        )

    fl8 = (
        jax.lax.broadcasted_iota(jnp.int32, (8, w), 0) * w
        + jax.lax.broadcasted_iota(jnp.int32, (8, w), 1)
    )
    thrs = []
    for r in range(_ROWS):
        idx = jnp.minimum(cnts[r] + int((1.0 - _TRIM) * n), n - 1)
        base = pl.multiple_of((idx // w) & ~7, 8)
        p8 = pred_ref[r, pl.ds(base, 8), :]
        t8 = tgt_ref[r, pl.ds(base, 8), :]
        r8 = jnp.where(t8 > 0, jnp.abs(p8 - t8), jnp.zeros_like(p8))
        thrs.append(
            jnp.sum(jnp.where(fl8 == idx - base * w, r8, jnp.zeros_like(r8)))
        )

    for r in range(_ROWS):
        tgt = tgt_ref[r]
        pred = pred_ref[r]
        d = jnp.abs(pred - tgt)
        keep = jnp.logical_and(tgt > 0, d <= thrs[r])
        s = _tree_sum(jnp.where(keep, d, jnp.zeros_like(d)))
        cnt_pos = n - cnts[r]
        valid = cnt_pos > 0
        denom = jnp.where(valid, 2 * cnt_pos, 1).astype(jnp.float32)
        loss_sum = loss_sum + jnp.where(valid, s / denom, 0.0)

    acc_ref[0] = loss_sum

    @pl.when(b == nb - 1)
    def _fin():
        out_ref[0] = loss_sum / (nb * _ROWS)


@functools.partial(jax.jit, static_argnames=())
def kernel(prediction, target):
    B, H, W = prediction.shape
    out = pl.pallas_call(
        _row_kernel,
        grid=(B // _ROWS,),
        in_specs=[
            pl.BlockSpec((_ROWS, H, W), lambda b: (b, 0, 0)),
            pl.BlockSpec((_ROWS, H, W), lambda b: (b, 0, 0)),
        ],
        out_specs=pl.BlockSpec(memory_space=pltpu.SMEM),
        out_shape=jax.ShapeDtypeStruct((1,), jnp.float32),
        scratch_shapes=[pltpu.SMEM((1,), jnp.float32)],
    )(prediction, target)
    return out[0]


# speedup vs baseline: 1.0139x; 1.0139x over previous
import functools

import jax
import jax.numpy as jnp
from jax.experimental import pallas as pl
from jax.experimental.pallas import tpu as pltpu

_TRIM = 0.2


_ROWS = 8


def _tree_sum(x):
    ones8 = jnp.full((8, x.shape[0]), 0.125, jnp.float32)
    partial = jax.lax.dot_general(
        ones8, x, (((1,), (0,)), ((), ())),
        preferred_element_type=jnp.float32,
    )
    return jnp.sum(partial)


def _row_kernel(pred_ref, tgt_ref, out_ref, acc_ref):
    b = pl.program_id(0)
    nb = pl.num_programs(0)
    h = pred_ref.shape[1]
    w = pred_ref.shape[2]
    n = h * w

    @pl.when(b == 0)
    def _init():
        acc_ref[0] = 0.0

    loss_sum = acc_ref[0]
    cnts = []
    for r in range(_ROWS):
        cnts.append(
            _tree_sum((tgt_ref[r] <= 0).astype(jnp.float32)).astype(jnp.int32)
        )

    fl8 = (
        jax.lax.broadcasted_iota(jnp.int32, (8, w), 0) * w
        + jax.lax.broadcasted_iota(jnp.int32, (8, w), 1)
    )
    thrs = []
    for r in range(_ROWS):
        idx = jnp.minimum(cnts[r] + int((1.0 - _TRIM) * n), n - 1)
        base = pl.multiple_of((idx // w) & ~7, 8)
        p8 = pred_ref[r, pl.ds(base, 8), :]
        t8 = tgt_ref[r, pl.ds(base, 8), :]
        r8 = jnp.where(t8 > 0, jnp.abs(p8 - t8), jnp.zeros_like(p8))
        thrs.append(
            jnp.sum(jnp.where(fl8 == idx - base * w, r8, jnp.zeros_like(r8)))
        )

    for r in range(_ROWS):
        tgt = tgt_ref[r]
        pred = pred_ref[r]
        d = jnp.abs(pred - tgt)
        keep = jnp.logical_and(tgt > 0, d <= thrs[r])
        s = _tree_sum(jnp.where(keep, d, jnp.zeros_like(d)))
        cnt_pos = n - cnts[r]
        valid = cnt_pos > 0
        denom = jnp.where(valid, 2 * cnt_pos, 1).astype(jnp.float32)
        loss_sum = loss_sum + jnp.where(valid, s / denom, 0.0)

    acc_ref[0] = loss_sum

    @pl.when(b == nb - 1)
    def _fin():
        out_ref[0] = loss_sum / (nb * _ROWS)


@functools.partial(jax.jit, static_argnames=())
def kernel(prediction, target):
    B, H, W = prediction.shape
    out = pl.pallas_call(
        _row_kernel,
        grid=(B // _ROWS,),
        in_specs=[
            pl.BlockSpec((_ROWS, H, W), lambda b: (b, 0, 0)),
            pl.BlockSpec((_ROWS, H, W), lambda b: (b, 0, 0)),
        ],
        out_specs=pl.BlockSpec(memory_space=pltpu.SMEM),
        out_shape=jax.ShapeDtypeStruct((1,), jnp.float32),
        scratch_shapes=[pltpu.SMEM((1,), jnp.float32)],
    )(prediction, target)
    return out[0]
